# Initial kernel scaffold; baseline (speedup 1.0000x reference)
import jax
import jax.numpy as jnp
from jax.experimental import pallas as pl


def kernel(x, edge_index, W1, b1, W2a, b2a, W2b, b2b):
    raise NotImplementedError("write your pallas kernel here")


# speedup vs baseline: 3.8905x; 3.8905x over previous
import jax
import jax.numpy as jnp
from jax import lax
from jax.experimental import pallas as pl
from jax.experimental.pallas import tpu as pltpu
from jax.experimental.pallas import tpu_sc as plsc

N_NODES = 10000
D_FEAT = 128
D_HALF = 64
D_HID = 128
D_OUT = 64
N_EDGES = 320000

NC = 2
NS = 16

CHUNK = 128
NCHUNK = 160
NCHUNK_HALF = NCHUNK // NC
EDGES_PER_TILE = CHUNK * NCHUNK
E_PAD = EDGES_PER_TILE * NS

N_PAD = 10240
ROWS_PER_TILE = N_PAD // NS
DEGW = 16


def _make_conv(with_deg: bool):
    mesh = plsc.VectorSubcoreMesh(core_axis_name="c", subcore_axis_name="s")
    out_type = [jax.ShapeDtypeStruct((N_PAD, D_HALF), jnp.float32),
                jax.ShapeDtypeStruct((N_PAD, D_HALF), jnp.float32)]
    scratch = [
        pltpu.VMEM((NCHUNK, CHUNK), jnp.int32),
        pltpu.VMEM((NCHUNK, CHUNK), jnp.int32),
        pltpu.VMEM((CHUNK, D_HALF), jnp.float32),
        pltpu.VMEM_SHARED((N_PAD, D_HALF), jnp.float32),
        pltpu.SemaphoreType.DMA,
    ]
    if with_deg:
        out_type.append(jax.ShapeDtypeStruct((NC, N_PAD, DEGW), jnp.float32))
        scratch.append(pltpu.VMEM((CHUNK, DEGW), jnp.float32))
        scratch.append(pltpu.VMEM_SHARED((N_PAD, DEGW), jnp.float32))

    def body(*refs):
        if with_deg:
            (xl_hbm, xr_hbm, src_hbm, dst_hbm, zagg_hbm, zdeg_hbm, ones_hbm,
             outl, outr, out_deg,
             src_v, dst_v, gbuf, agg_sh, sem, ones_v, deg_sh) = refs
        else:
            (xl_hbm, xr_hbm, src_hbm, dst_hbm, zagg_hbm,
             outl, outr,
             src_v, dst_v, gbuf, agg_sh, sem) = refs
        cid = lax.axis_index("c")
        sid = lax.axis_index("s")
        rows = pl.ds(sid * ROWS_PER_TILE, ROWS_PER_TILE)
        pltpu.sync_copy(zagg_hbm.at[rows], agg_sh.at[rows])
        if with_deg:
            pltpu.sync_copy(zdeg_hbm.at[rows], deg_sh.at[rows])
            pltpu.sync_copy(ones_hbm, ones_v)
        pltpu.sync_copy(src_hbm.at[sid], src_v)
        pltpu.sync_copy(dst_hbm.at[sid], dst_v)
        plsc.subcore_barrier()

        def make_step(table):
            def step(j, carry):
                pltpu.async_copy(table.at[src_v.at[j]], gbuf, sem).wait()
                pltpu.sync_copy(gbuf, agg_sh.at[dst_v.at[j]], add=True)
                if with_deg:
                    @pl.when((j >= cid * NCHUNK_HALF)
                             & (j < (cid + 1) * NCHUNK_HALF))
                    def _():
                        pltpu.sync_copy(ones_v, deg_sh.at[dst_v.at[j]],
                                        add=True)
                return carry
            return step

        @pl.when(cid == 0)
        def _():
            lax.fori_loop(0, NCHUNK, make_step(xl_hbm), 0)

        @pl.when(cid == 1)
        def _():
            lax.fori_loop(0, NCHUNK, make_step(xr_hbm), 0)

        plsc.subcore_barrier()

        @pl.when(cid == 0)
        def _():
            pltpu.sync_copy(agg_sh.at[rows], outl.at[rows])

        @pl.when(cid == 1)
        def _():
            pltpu.sync_copy(agg_sh.at[rows], outr.at[rows])

        if with_deg:
            pltpu.sync_copy(deg_sh.at[rows], out_deg.at[cid, rows])

    return pl.kernel(body, mesh=mesh, out_type=out_type,
                     scratch_types=scratch,
                     compiler_params=pltpu.CompilerParams(
                         use_tc_tiling_on_sc=False))


_conv_deg = _make_conv(True)
_conv = _make_conv(False)

_R = 1000


def _tc1_body(al, ar, d0, d1, w1, b1, ol, orr):
    deg = jnp.maximum(d0[:, 0:1] + d1[:, 0:1], 1.0)
    agg = jnp.concatenate([al[...], ar[...]], axis=1) / deg
    h = jnp.dot(agg, w1[...], preferred_element_type=jnp.float32,
                precision=lax.Precision.HIGHEST) + b1[...]
    h = jnp.maximum(h, 0.0)
    n2 = jnp.sum(h * h, axis=1, keepdims=True)
    h = h * lax.rsqrt(jnp.maximum(n2, 1e-24))
    ol[...] = h[:, :D_HALF]
    orr[...] = h[:, D_HALF:]


def _tc2_body(al, ar, d0, d1, w2a, b2a, w2b, b2b, o):
    deg = jnp.maximum(d0[:, 0:1] + d1[:, 0:1], 1.0)
    agg = jnp.concatenate([al[...], ar[...]], axis=1) / deg
    h = jnp.dot(agg, w2a[...], preferred_element_type=jnp.float32,
                precision=lax.Precision.HIGHEST) + b2a[...]
    h = jnp.maximum(h, 0.0)
    h = jnp.dot(h, w2b[...], preferred_element_type=jnp.float32,
                precision=lax.Precision.HIGHEST) + b2b[...]
    n2 = jnp.sum(h * h, axis=1, keepdims=True)
    o[...] = h * lax.rsqrt(jnp.maximum(n2, 1e-24))


def _row_spec(d):
    return pl.BlockSpec((_R, d), lambda i: (i, 0))


def _full_spec(r, d):
    return pl.BlockSpec((r, d), lambda i: (0, 0))


_tc1 = pl.pallas_call(
    _tc1_body,
    grid=(N_NODES // _R,),
    in_specs=[_row_spec(D_HALF), _row_spec(D_HALF),
              _row_spec(DEGW), _row_spec(DEGW),
              _full_spec(D_FEAT, D_FEAT), _full_spec(1, D_FEAT)],
    out_specs=[_row_spec(D_HALF), _row_spec(D_HALF)],
    out_shape=[jax.ShapeDtypeStruct((N_NODES, D_HALF), jnp.float32),
               jax.ShapeDtypeStruct((N_NODES, D_HALF), jnp.float32)],
)

_tc2 = pl.pallas_call(
    _tc2_body,
    grid=(N_NODES // _R,),
    in_specs=[_row_spec(D_HALF), _row_spec(D_HALF),
              _row_spec(DEGW), _row_spec(DEGW),
              _full_spec(D_FEAT, D_HID), _full_spec(1, D_HID),
              _full_spec(D_HID, D_OUT), _full_spec(1, D_OUT)],
    out_specs=_row_spec(D_OUT),
    out_shape=jax.ShapeDtypeStruct((N_NODES, D_OUT), jnp.float32),
)


def _pad_half(a):
    return jnp.pad(a, ((0, N_PAD - N_NODES), (0, 0)))


def kernel(x, edge_index, W1, b1, W2a, b2a, W2b, b2b):
    src = edge_index[0].astype(jnp.int32)
    dst = edge_index[1].astype(jnp.int32)
    pad = E_PAD - N_EDGES
    src_p = jnp.concatenate(
        [src, jnp.zeros((pad,), jnp.int32)]).reshape(NS, NCHUNK, CHUNK)
    dst_p = jnp.concatenate(
        [dst, jnp.full((pad,), N_NODES, jnp.int32)]).reshape(NS, NCHUNK, CHUNK)
    zagg = jnp.zeros((N_PAD, D_HALF), jnp.float32)
    zdeg = jnp.zeros((N_PAD, DEGW), jnp.float32)
    ones = jnp.ones((CHUNK, DEGW), jnp.float32)

    xl = x[:, :D_HALF]
    xr = x[:, D_HALF:]
    aggl, aggr, deg_p = _conv_deg(xl, xr, src_p, dst_p, zagg, zdeg, ones)
    d0 = deg_p[0, :N_NODES]
    d1 = deg_p[1, :N_NODES]
    hl, hr = _tc1(aggl[:N_NODES], aggr[:N_NODES], d0, d1,
                  W1, b1.reshape(1, D_FEAT))
    aggl2, aggr2 = _conv(hl, hr, src_p, dst_p, zagg)
    out = _tc2(aggl2[:N_NODES], aggr2[:N_NODES], d0, d1,
               W2a, b2a.reshape(1, D_HID), W2b, b2b.reshape(1, D_OUT))
    return out


# speedup vs baseline: 4.4717x; 1.1494x over previous
import jax
import jax.numpy as jnp
from jax import lax
from jax.experimental import pallas as pl
from jax.experimental.pallas import tpu as pltpu
from jax.experimental.pallas import tpu_sc as plsc

N_NODES = 10000
D_FEAT = 128
D_HALF = 64
D_HID = 128
D_OUT = 64
N_EDGES = 320000

NC = 2
NS = 16

CHUNK = 128
NCHUNK = 160
NCHUNK_HALF = NCHUNK // NC
EDGES_PER_TILE = CHUNK * NCHUNK
E_PAD = EDGES_PER_TILE * NS

N_PAD = 10240
ROWS_PER_TILE = N_PAD // NS
DEGW = 16


def _make_conv(with_deg: bool):
    mesh = plsc.VectorSubcoreMesh(core_axis_name="c", subcore_axis_name="s")
    out_type = [jax.ShapeDtypeStruct((N_PAD, D_HALF), jnp.float32),
                jax.ShapeDtypeStruct((N_PAD, D_HALF), jnp.float32)]
    scratch = [
        pltpu.VMEM((NCHUNK, CHUNK), jnp.int32),
        pltpu.VMEM((NCHUNK, CHUNK), jnp.int32),
        pltpu.VMEM((2, CHUNK, D_HALF), jnp.float32),
        pltpu.VMEM_SHARED((N_PAD, D_HALF), jnp.float32),
        pltpu.SemaphoreType.DMA,
        pltpu.SemaphoreType.DMA,
    ]
    if with_deg:
        out_type.append(jax.ShapeDtypeStruct((NC, N_PAD, DEGW), jnp.float32))
        scratch.append(pltpu.VMEM((CHUNK, DEGW), jnp.float32))
        scratch.append(pltpu.VMEM_SHARED((N_PAD, DEGW), jnp.float32))
        scratch.append(pltpu.SemaphoreType.DMA)

    def body(*refs):
        if with_deg:
            (xl_hbm, xr_hbm, src_hbm, dst_hbm, zagg_hbm, zdeg_hbm, ones_hbm,
             outl, outr, out_deg,
             src_v, dst_v, gbuf, agg_sh, sem_g, sem_s,
             ones_v, deg_sh, sem_d) = refs
        else:
            (xl_hbm, xr_hbm, src_hbm, dst_hbm, zagg_hbm,
             outl, outr,
             src_v, dst_v, gbuf, agg_sh, sem_g, sem_s) = refs
        cid = lax.axis_index("c")
        sid = lax.axis_index("s")
        rows = pl.ds(sid * ROWS_PER_TILE, ROWS_PER_TILE)
        pltpu.sync_copy(zagg_hbm.at[rows], agg_sh.at[rows])
        if with_deg:
            pltpu.sync_copy(zdeg_hbm.at[rows], deg_sh.at[rows])
            pltpu.sync_copy(ones_hbm, ones_v)
        pltpu.sync_copy(src_hbm.at[sid], src_v)
        pltpu.sync_copy(dst_hbm.at[sid], dst_v)
        plsc.subcore_barrier()

        lo = cid * NCHUNK_HALF
        hi = lo + NCHUNK_HALF

        def make_loop(table):
            def wait_gather(b):
                pltpu.make_async_copy(
                    table.at[src_v.at[0]], gbuf.at[b], sem_g).wait()

            def wait_scatter(b):
                pltpu.make_async_copy(
                    gbuf.at[b], agg_sh.at[pl.ds(0, CHUNK)], sem_s).wait()

            def step(w, carry):
                for b in range(2):
                    i = 2 * w + b
                    nb = 1 - b
                    wait_gather(b)

                    @pl.when(i >= 1)
                    def _():
                        wait_scatter(nb)

                    @pl.when(i <= NCHUNK - 2)
                    def _():
                        pltpu.async_copy(
                            table.at[src_v.at[i + 1]], gbuf.at[nb], sem_g)

                    pltpu.async_copy(
                        gbuf.at[b], agg_sh.at[dst_v.at[i]], sem_s, add=True)
                    if with_deg:
                        @pl.when((i > lo) & (i < hi))
                        def _():
                            pltpu.make_async_copy(
                                ones_v, deg_sh.at[pl.ds(0, CHUNK)],
                                sem_d).wait()

                        @pl.when((i >= lo) & (i < hi))
                        def _():
                            pltpu.async_copy(
                                ones_v, deg_sh.at[dst_v.at[i]], sem_d,
                                add=True)
                return carry

            pltpu.async_copy(table.at[src_v.at[0]], gbuf.at[0], sem_g)
            lax.fori_loop(0, NCHUNK // 2, step, 0)
            wait_scatter(1)
            if with_deg:
                pltpu.make_async_copy(
                    ones_v, deg_sh.at[pl.ds(0, CHUNK)], sem_d).wait()

        @pl.when(cid == 0)
        def _():
            make_loop(xl_hbm)

        @pl.when(cid == 1)
        def _():
            make_loop(xr_hbm)

        plsc.subcore_barrier()

        @pl.when(cid == 0)
        def _():
            pltpu.sync_copy(agg_sh.at[rows], outl.at[rows])

        @pl.when(cid == 1)
        def _():
            pltpu.sync_copy(agg_sh.at[rows], outr.at[rows])

        if with_deg:
            pltpu.sync_copy(deg_sh.at[rows], out_deg.at[cid, rows])

    return pl.kernel(body, mesh=mesh, out_type=out_type,
                     scratch_types=scratch,
                     compiler_params=pltpu.CompilerParams(
                         use_tc_tiling_on_sc=False))


_conv_deg = _make_conv(True)
_conv = _make_conv(False)

_R = 1000


def _tc1_body(al, ar, d0, d1, w1, b1, ol, orr):
    deg = jnp.maximum(d0[:, 0:1] + d1[:, 0:1], 1.0)
    agg = jnp.concatenate([al[...], ar[...]], axis=1) / deg
    h = jnp.dot(agg, w1[...], preferred_element_type=jnp.float32,
                precision=lax.Precision.HIGHEST) + b1[...]
    h = jnp.maximum(h, 0.0)
    n2 = jnp.sum(h * h, axis=1, keepdims=True)
    h = h * lax.rsqrt(jnp.maximum(n2, 1e-24))
    ol[...] = h[:, :D_HALF]
    orr[...] = h[:, D_HALF:]


def _tc2_body(al, ar, d0, d1, w2a, b2a, w2b, b2b, o):
    deg = jnp.maximum(d0[:, 0:1] + d1[:, 0:1], 1.0)
    agg = jnp.concatenate([al[...], ar[...]], axis=1) / deg
    h = jnp.dot(agg, w2a[...], preferred_element_type=jnp.float32,
                precision=lax.Precision.HIGHEST) + b2a[...]
    h = jnp.maximum(h, 0.0)
    h = jnp.dot(h, w2b[...], preferred_element_type=jnp.float32,
                precision=lax.Precision.HIGHEST) + b2b[...]
    n2 = jnp.sum(h * h, axis=1, keepdims=True)
    o[...] = h * lax.rsqrt(jnp.maximum(n2, 1e-24))


def _row_spec(d):
    return pl.BlockSpec((_R, d), lambda i: (i, 0))


def _full_spec(r, d):
    return pl.BlockSpec((r, d), lambda i: (0, 0))


_tc1 = pl.pallas_call(
    _tc1_body,
    grid=(N_NODES // _R,),
    in_specs=[_row_spec(D_HALF), _row_spec(D_HALF),
              _row_spec(DEGW), _row_spec(DEGW),
              _full_spec(D_FEAT, D_FEAT), _full_spec(1, D_FEAT)],
    out_specs=[_row_spec(D_HALF), _row_spec(D_HALF)],
    out_shape=[jax.ShapeDtypeStruct((N_NODES, D_HALF), jnp.float32),
               jax.ShapeDtypeStruct((N_NODES, D_HALF), jnp.float32)],
)

_tc2 = pl.pallas_call(
    _tc2_body,
    grid=(N_NODES // _R,),
    in_specs=[_row_spec(D_HALF), _row_spec(D_HALF),
              _row_spec(DEGW), _row_spec(DEGW),
              _full_spec(D_FEAT, D_HID), _full_spec(1, D_HID),
              _full_spec(D_HID, D_OUT), _full_spec(1, D_OUT)],
    out_specs=_row_spec(D_OUT),
    out_shape=jax.ShapeDtypeStruct((N_NODES, D_OUT), jnp.float32),
)


def _pad_half(a):
    return jnp.pad(a, ((0, N_PAD - N_NODES), (0, 0)))


def kernel(x, edge_index, W1, b1, W2a, b2a, W2b, b2b):
    src = edge_index[0].astype(jnp.int32)
    dst = edge_index[1].astype(jnp.int32)
    pad = E_PAD - N_EDGES
    src_p = jnp.concatenate(
        [src, jnp.zeros((pad,), jnp.int32)]).reshape(NS, NCHUNK, CHUNK)
    dst_p = jnp.concatenate(
        [dst, jnp.full((pad,), N_NODES, jnp.int32)]).reshape(NS, NCHUNK, CHUNK)
    zagg = jnp.zeros((N_PAD, D_HALF), jnp.float32)
    zdeg = jnp.zeros((N_PAD, DEGW), jnp.float32)
    ones = jnp.ones((CHUNK, DEGW), jnp.float32)

    xl = x[:, :D_HALF]
    xr = x[:, D_HALF:]
    aggl, aggr, deg_p = _conv_deg(xl, xr, src_p, dst_p, zagg, zdeg, ones)
    d0 = deg_p[0, :N_NODES]
    d1 = deg_p[1, :N_NODES]
    hl, hr = _tc1(aggl[:N_NODES], aggr[:N_NODES], d0, d1,
                  W1, b1.reshape(1, D_FEAT))
    aggl2, aggr2 = _conv(hl, hr, src_p, dst_p, zagg)
    out = _tc2(aggl2[:N_NODES], aggr2[:N_NODES], d0, d1,
               W2a, b2a.reshape(1, D_HID), W2b, b2b.reshape(1, D_OUT))
    return out
